# baseline (speedup 1.0000x reference)
import functools

import jax
import jax.numpy as jnp
from jax import lax
from jax.experimental import pallas as pl
from jax.experimental.pallas import tpu as pltpu
from jax.experimental.pallas import tpu_sc as plsc

N = 10000
D = 128
E = 320000

BLK = 400
EBK = 80
CH = 128
NCH = 160
EPT = CH * NCH
EP = 16 * EPT
RPT = 640
CPR = 40
ACC_N = 16 * RPT


def _prologue_body(x_ref, pos_ref, win_ref, wsrc_ref, wlin_ref, wpos_ref,
                   bin_ref, bpos_ref, table_ref, q_ref):
    h = jax.nn.relu(
        jnp.dot(x_ref[...], win_ref[...], preferred_element_type=jnp.float32)
        + bin_ref[...])
    p = jnp.dot(pos_ref[...], wpos_ref[...], preferred_element_type=jnp.float32)
    b = jnp.dot(h, wsrc_ref[...], preferred_element_type=jnp.float32) + p
    c = jnp.dot(h, wlin_ref[...], preferred_element_type=jnp.float32) - p
    g = jnp.exp(-b)
    table_ref[...] = jnp.concatenate([g, g * c], axis=1).reshape(BLK, 2 * D)
    q_ref[...] = p + bpos_ref[...]


def _epilogue_body(t0_ref, t1_ref, q_ref, wout_ref, bout_ref, o_ref):
    t0 = t0_ref[...]
    agg = (t1_ref[...] + q_ref[...] * t0) / (t0 + 1e-16)
    o_ref[...] = jax.nn.relu(
        jnp.dot(agg, wout_ref[...], preferred_element_type=jnp.float32)
        + bout_ref[...])


def _sc_edge_body(table_hbm, src_hbm, dst_hbm, zeros_hbm, out_hbm,
                  sidx, gidx, didx, rows, buf, acc, sem):
    cid = lax.axis_index("c")
    sid = lax.axis_index("s")

    pltpu.sync_copy(zeros_hbm, buf)
    for k in range(RPT // CPR):
        pltpu.sync_copy(buf, acc.at[pl.ds(sid * RPT + k * CPR, CPR)])
    plsc.subcore_barrier()

    ebase = sid * EPT

    def body(i, carry):
        base = ebase + i * CH
        pltpu.sync_copy(src_hbm.at[pl.ds(base, CH)], sidx)
        pltpu.sync_copy(dst_hbm.at[pl.ds(base, CH)], didx)
        for j in range(CH // 16):
            s = sidx[pl.ds(j * 16, 16)]
            gidx[pl.ds(j * 16, 16)] = s * 2 + cid
        pltpu.async_copy(table_hbm.at[gidx], rows, sem).wait()
        pltpu.sync_copy(rows, acc.at[didx], add=True)
        return carry

    lax.fori_loop(0, NCH, body, 0)
    plsc.subcore_barrier()

    for k in range(RPT // CPR):
        pltpu.sync_copy(acc.at[pl.ds(sid * RPT + k * CPR, CPR)], buf)
        pltpu.sync_copy(buf, out_hbm.at[pl.ds(cid * ACC_N + sid * RPT + k * CPR, CPR)])


@functools.partial(
    pl.kernel,
    out_type=jax.ShapeDtypeStruct((2 * ACC_N, D), jnp.float32),
    mesh=plsc.VectorSubcoreMesh(core_axis_name="c", subcore_axis_name="s"),
    scratch_types=[
        pltpu.VMEM((CH,), jnp.int32),
        pltpu.VMEM((CH,), jnp.int32),
        pltpu.VMEM((CH,), jnp.int32),
        pltpu.VMEM((CH, D), jnp.float32),
        pltpu.VMEM((CPR, D), jnp.float32),
        pltpu.VMEM_SHARED((ACC_N, D), jnp.float32),
        pltpu.SemaphoreType.DMA,
    ],
)
def _sc_edge_kernel(table_hbm, src_hbm, dst_hbm, zeros_hbm, out_hbm,
                    sidx, gidx, didx, rows, buf, acc, sem):
    _sc_edge_body(table_hbm, src_hbm, dst_hbm, zeros_hbm, out_hbm,
                  sidx, gidx, didx, rows, buf, acc, sem)


def kernel(x, edge_index, pos, W_in, b_in, W_lin, W_src, W_dst, W_pos, b_pos,
           W_out, b_out):
    del W_dst

    posp = jnp.pad(pos, ((0, 0), (0, 8 - pos.shape[1])))
    wpos_t = jnp.pad(W_pos.T, ((0, 8 - W_pos.shape[1]), (0, 0)))

    table, q = pl.pallas_call(
        _prologue_body,
        grid=(N // BLK,),
        in_specs=[
            pl.BlockSpec((BLK, D), lambda i: (i, 0)),
            pl.BlockSpec((BLK, 8), lambda i: (i, 0)),
            pl.BlockSpec((D, D), lambda i: (0, 0)),
            pl.BlockSpec((D, D), lambda i: (0, 0)),
            pl.BlockSpec((D, D), lambda i: (0, 0)),
            pl.BlockSpec((8, D), lambda i: (0, 0)),
            pl.BlockSpec((1, D), lambda i: (0, 0)),
            pl.BlockSpec((1, D), lambda i: (0, 0)),
        ],
        out_specs=[
            pl.BlockSpec((BLK, 2 * D), lambda i: (i, 0)),
            pl.BlockSpec((BLK, D), lambda i: (i, 0)),
        ],
        out_shape=[
            jax.ShapeDtypeStruct((N, 2 * D), jnp.float32),
            jax.ShapeDtypeStruct((N, D), jnp.float32),
        ],
    )(x, posp, W_in.T, W_src.T, W_lin.T, wpos_t, b_in.reshape(1, D),
      b_pos.reshape(1, D))

    table2 = table.reshape(2 * N, D)

    src = edge_index[0].astype(jnp.int32)
    dst = edge_index[1].astype(jnp.int32)
    srcp = jnp.pad(src, (0, EP - E))
    dstp = jnp.pad(dst, (0, EP - E), constant_values=N)
    zeros = jnp.zeros((CPR, D), jnp.float32)

    sc_out = _sc_edge_kernel(table2, srcp, dstp, zeros)

    out = pl.pallas_call(
        _epilogue_body,
        grid=(N // EBK,),
        in_specs=[
            pl.BlockSpec((EBK, D), lambda i: (i, 0)),
            pl.BlockSpec((EBK, D), lambda i: (i + ACC_N // EBK, 0)),
            pl.BlockSpec((EBK, D), lambda i: (i, 0)),
            pl.BlockSpec((D, D), lambda i: (0, 0)),
            pl.BlockSpec((1, D), lambda i: (0, 0)),
        ],
        out_specs=pl.BlockSpec((EBK, D), lambda i: (i, 0)),
        out_shape=jax.ShapeDtypeStruct((N, D), jnp.float32),
    )(sc_out, sc_out, q, W_out.T, b_out.reshape(1, D))

    return out


# speedup vs baseline: 1.0010x; 1.0010x over previous
import functools

import jax
import jax.numpy as jnp
from jax import lax
from jax.experimental import pallas as pl
from jax.experimental.pallas import tpu as pltpu
from jax.experimental.pallas import tpu_sc as plsc

N = 10000
D = 128
E = 320000

BLK = 400
EBK = 80
CH = 128
NCH = 160
EPT = CH * NCH
EP = 16 * EPT
RPT = 640
CPR = 160
ACC_N = 16 * RPT


def _prologue_body(x_ref, pos_ref, win_ref, wsrc_ref, wlin_ref, wpos_ref,
                   bin_ref, bpos_ref, table_ref, q_ref):
    h = jax.nn.relu(
        jnp.dot(x_ref[...], win_ref[...], preferred_element_type=jnp.float32)
        + bin_ref[...])
    p = jnp.dot(pos_ref[...], wpos_ref[...], preferred_element_type=jnp.float32)
    b = jnp.dot(h, wsrc_ref[...], preferred_element_type=jnp.float32) + p
    c = jnp.dot(h, wlin_ref[...], preferred_element_type=jnp.float32) - p
    g = jnp.exp(-b)
    table_ref[...] = jnp.concatenate([g, g * c], axis=1).reshape(BLK, 2 * D)
    q_ref[...] = p + bpos_ref[...]


def _epilogue_body(t0_ref, t1_ref, q_ref, wout_ref, bout_ref, o_ref):
    t0 = t0_ref[...]
    agg = (t1_ref[...] + q_ref[...] * t0) / (t0 + 1e-16)
    o_ref[...] = jax.nn.relu(
        jnp.dot(agg, wout_ref[...], preferred_element_type=jnp.float32)
        + bout_ref[...])


def _sc_edge_body(table_hbm, src_hbm, dst_hbm, zeros_hbm, out_hbm,
                  sidx, gidx, didx, rows, buf, acc, sem):
    cid = lax.axis_index("c")
    sid = lax.axis_index("s")

    pltpu.sync_copy(zeros_hbm, buf)
    for k in range(RPT // CPR):
        pltpu.sync_copy(buf, acc.at[pl.ds(sid * RPT + k * CPR, CPR)])
    plsc.subcore_barrier()

    ebase = sid * EPT

    def body(i, carry):
        base = ebase + i * CH
        pltpu.sync_copy(src_hbm.at[pl.ds(base, CH)], sidx)
        pltpu.sync_copy(dst_hbm.at[pl.ds(base, CH)], didx)
        for j in range(CH // 16):
            s = sidx[pl.ds(j * 16, 16)]
            gidx[pl.ds(j * 16, 16)] = s * 2 + cid
        pltpu.async_copy(table_hbm.at[gidx], rows, sem).wait()
        pltpu.sync_copy(rows, acc.at[didx], add=True)
        return carry

    lax.fori_loop(0, NCH, body, 0)
    plsc.subcore_barrier()

    for k in range(RPT // CPR):
        pltpu.sync_copy(acc.at[pl.ds(sid * RPT + k * CPR, CPR)], buf)
        pltpu.sync_copy(buf, out_hbm.at[pl.ds(cid * ACC_N + sid * RPT + k * CPR, CPR)])


@functools.partial(
    pl.kernel,
    out_type=jax.ShapeDtypeStruct((2 * ACC_N, D), jnp.float32),
    mesh=plsc.VectorSubcoreMesh(core_axis_name="c", subcore_axis_name="s"),
    scratch_types=[
        pltpu.VMEM((CH,), jnp.int32),
        pltpu.VMEM((CH,), jnp.int32),
        pltpu.VMEM((CH,), jnp.int32),
        pltpu.VMEM((CH, D), jnp.float32),
        pltpu.VMEM((CPR, D), jnp.float32),
        pltpu.VMEM_SHARED((ACC_N, D), jnp.float32),
        pltpu.SemaphoreType.DMA,
    ],
)
def _sc_edge_kernel(table_hbm, src_hbm, dst_hbm, zeros_hbm, out_hbm,
                    sidx, gidx, didx, rows, buf, acc, sem):
    _sc_edge_body(table_hbm, src_hbm, dst_hbm, zeros_hbm, out_hbm,
                  sidx, gidx, didx, rows, buf, acc, sem)


def kernel(x, edge_index, pos, W_in, b_in, W_lin, W_src, W_dst, W_pos, b_pos,
           W_out, b_out):
    del W_dst

    posp = jnp.pad(pos, ((0, 0), (0, 8 - pos.shape[1])))
    wpos_t = jnp.pad(W_pos.T, ((0, 8 - W_pos.shape[1]), (0, 0)))

    table, q = pl.pallas_call(
        _prologue_body,
        grid=(N // BLK,),
        in_specs=[
            pl.BlockSpec((BLK, D), lambda i: (i, 0)),
            pl.BlockSpec((BLK, 8), lambda i: (i, 0)),
            pl.BlockSpec((D, D), lambda i: (0, 0)),
            pl.BlockSpec((D, D), lambda i: (0, 0)),
            pl.BlockSpec((D, D), lambda i: (0, 0)),
            pl.BlockSpec((8, D), lambda i: (0, 0)),
            pl.BlockSpec((1, D), lambda i: (0, 0)),
            pl.BlockSpec((1, D), lambda i: (0, 0)),
        ],
        out_specs=[
            pl.BlockSpec((BLK, 2 * D), lambda i: (i, 0)),
            pl.BlockSpec((BLK, D), lambda i: (i, 0)),
        ],
        out_shape=[
            jax.ShapeDtypeStruct((N, 2 * D), jnp.float32),
            jax.ShapeDtypeStruct((N, D), jnp.float32),
        ],
    )(x, posp, W_in.T, W_src.T, W_lin.T, wpos_t, b_in.reshape(1, D),
      b_pos.reshape(1, D))

    table2 = table.reshape(2 * N, D)

    src = edge_index[0].astype(jnp.int32)
    dst = edge_index[1].astype(jnp.int32)
    srcp = jnp.pad(src, (0, EP - E))
    dstp = jnp.pad(dst, (0, EP - E), constant_values=N)
    zeros = jnp.zeros((CPR, D), jnp.float32)

    sc_out = _sc_edge_kernel(table2, srcp, dstp, zeros)

    out = pl.pallas_call(
        _epilogue_body,
        grid=(N // EBK,),
        in_specs=[
            pl.BlockSpec((EBK, D), lambda i: (i, 0)),
            pl.BlockSpec((EBK, D), lambda i: (i + ACC_N // EBK, 0)),
            pl.BlockSpec((EBK, D), lambda i: (i, 0)),
            pl.BlockSpec((D, D), lambda i: (0, 0)),
            pl.BlockSpec((1, D), lambda i: (0, 0)),
        ],
        out_specs=pl.BlockSpec((EBK, D), lambda i: (i, 0)),
        out_shape=jax.ShapeDtypeStruct((N, D), jnp.float32),
    )(sc_out, sc_out, q, W_out.T, b_out.reshape(1, D))

    return out


# speedup vs baseline: 1.0066x; 1.0056x over previous
import functools

import jax
import jax.numpy as jnp
from jax import lax
from jax.experimental import pallas as pl
from jax.experimental.pallas import tpu as pltpu
from jax.experimental.pallas import tpu_sc as plsc

N = 10000
D = 128
E = 320000

BLK = 400
EBK = 80
CH = 128
NCH = 160
EPT = CH * NCH
EP = 16 * EPT
RPT = 640
CPR = 160
ACC_N = 16 * RPT


def _prologue_body(x_ref, pos_ref, win_ref, wsrc_ref, wlin_ref, wpos_ref,
                   bin_ref, bpos_ref, table_ref, q_ref):
    h = jax.nn.relu(
        jnp.dot(x_ref[...], win_ref[...], preferred_element_type=jnp.float32)
        + bin_ref[...])
    p = jnp.dot(pos_ref[...], wpos_ref[...], preferred_element_type=jnp.float32)
    b = jnp.dot(h, wsrc_ref[...], preferred_element_type=jnp.float32) + p
    c = jnp.dot(h, wlin_ref[...], preferred_element_type=jnp.float32) - p
    g = jnp.exp(-b)
    table_ref[...] = jnp.concatenate([g, g * c], axis=1).reshape(BLK, 2 * D)
    q_ref[...] = p + bpos_ref[...]


def _epilogue_body(t0_ref, t1_ref, q_ref, wout_ref, bout_ref, o_ref):
    t0 = t0_ref[...]
    agg = (t1_ref[...] + q_ref[...] * t0) / (t0 + 1e-16)
    o_ref[...] = jax.nn.relu(
        jnp.dot(agg, wout_ref[...], preferred_element_type=jnp.float32)
        + bout_ref[...])


def _sc_edge_body(table_hbm, src_hbm, dst_hbm, zeros_hbm, out_hbm,
                  sidx, gidx, didx, rows, buf, acc, sem):
    cid = lax.axis_index("c")
    sid = lax.axis_index("s")

    pltpu.sync_copy(zeros_hbm, buf)
    for k in range(RPT // CPR):
        pltpu.sync_copy(buf, acc.at[pl.ds(sid * RPT + k * CPR, CPR)])
    plsc.subcore_barrier()

    ebase = sid * EPT

    def body(i, carry):
        base = ebase + i * CH
        pltpu.sync_copy(src_hbm.at[pl.ds(base, CH)], sidx)
        pltpu.sync_copy(dst_hbm.at[pl.ds(base, CH)], didx)
        for j in range(CH // 16):
            s = sidx[pl.ds(j * 16, 16)]
            gidx[pl.ds(j * 16, 16)] = s * 2 + cid
        pltpu.async_copy(table_hbm.at[gidx], rows, sem).wait()
        pltpu.sync_copy(rows, acc.at[didx], add=True)
        return carry

    lax.fori_loop(0, NCH, body, 0)
    plsc.subcore_barrier()

    for k in range(RPT // CPR):
        pltpu.sync_copy(acc.at[pl.ds(sid * RPT + k * CPR, CPR)], buf)
        pltpu.sync_copy(buf, out_hbm.at[pl.ds(cid * ACC_N + sid * RPT + k * CPR, CPR)])


@functools.partial(
    pl.kernel,
    out_type=jax.ShapeDtypeStruct((2 * ACC_N, D), jnp.float32),
    mesh=plsc.VectorSubcoreMesh(core_axis_name="c", subcore_axis_name="s"),
    scratch_types=[
        pltpu.VMEM((CH,), jnp.int32),
        pltpu.VMEM((CH,), jnp.int32),
        pltpu.VMEM((CH,), jnp.int32),
        pltpu.VMEM((CH, D), jnp.float32),
        pltpu.VMEM((CPR, D), jnp.float32),
        pltpu.VMEM_SHARED((ACC_N, D), jnp.float32),
        pltpu.SemaphoreType.DMA,
    ],
)
def _sc_edge_kernel(table_hbm, src_hbm, dst_hbm, zeros_hbm, out_hbm,
                    sidx, gidx, didx, rows, buf, acc, sem):
    _sc_edge_body(table_hbm, src_hbm, dst_hbm, zeros_hbm, out_hbm,
                  sidx, gidx, didx, rows, buf, acc, sem)


def kernel(x, edge_index, pos, W_in, b_in, W_lin, W_src, W_dst, W_pos, b_pos,
           W_out, b_out):
    del W_dst

    posp = jnp.pad(pos, ((0, 0), (0, 8 - pos.shape[1])))
    wpos_t = jnp.pad(W_pos.T, ((0, 8 - W_pos.shape[1]), (0, 0)))

    table, q = pl.pallas_call(
        _prologue_body,
        grid=(N // BLK,),
        in_specs=[
            pl.BlockSpec((BLK, D), lambda i: (i, 0)),
            pl.BlockSpec((BLK, 8), lambda i: (i, 0)),
            pl.BlockSpec((D, D), lambda i: (0, 0)),
            pl.BlockSpec((D, D), lambda i: (0, 0)),
            pl.BlockSpec((D, D), lambda i: (0, 0)),
            pl.BlockSpec((8, D), lambda i: (0, 0)),
            pl.BlockSpec((1, D), lambda i: (0, 0)),
            pl.BlockSpec((1, D), lambda i: (0, 0)),
        ],
        out_specs=[
            pl.BlockSpec((BLK, 2 * D), lambda i: (i, 0)),
            pl.BlockSpec((BLK, D), lambda i: (i, 0)),
        ],
        out_shape=[
            jax.ShapeDtypeStruct((N, 2 * D), jnp.float32),
            jax.ShapeDtypeStruct((N, D), jnp.float32),
        ],
    )(x, posp, W_in.T, W_src.T, W_lin.T, wpos_t, b_in.reshape(1, D),
      b_pos.reshape(1, D))

    table2 = table.reshape(2 * N, D)

    src = edge_index[0].astype(jnp.int32)
    dst = edge_index[1].astype(jnp.int32)
    srcp = jnp.pad(src, (0, EP - E))
    dstp = jnp.pad(dst, (0, EP - E), constant_values=N)
    zeros = jnp.zeros((CPR, D), jnp.float32)

    sc_out = _sc_edge_kernel(table2, srcp, dstp, zeros)
    t0 = sc_out[0:N]
    t1 = sc_out[ACC_N:ACC_N + N]

    out = pl.pallas_call(
        _epilogue_body,
        grid=(N // EBK,),
        in_specs=[
            pl.BlockSpec((EBK, D), lambda i: (i, 0)),
            pl.BlockSpec((EBK, D), lambda i: (i, 0)),
            pl.BlockSpec((EBK, D), lambda i: (i, 0)),
            pl.BlockSpec((D, D), lambda i: (0, 0)),
            pl.BlockSpec((1, D), lambda i: (0, 0)),
        ],
        out_specs=pl.BlockSpec((EBK, D), lambda i: (i, 0)),
        out_shape=jax.ShapeDtypeStruct((N, D), jnp.float32),
    )(t0, t1, q, W_out.T, b_out.reshape(1, D))

    return out


# speedup vs baseline: 1.0608x; 1.0538x over previous
import functools

import jax
import jax.numpy as jnp
from jax import lax
from jax.experimental import pallas as pl
from jax.experimental.pallas import tpu as pltpu
from jax.experimental.pallas import tpu_sc as plsc

N = 10000
D = 128
E = 320000

BLK = 400
EBK = 80
CH = 128
NCH = 160
EPT = CH * NCH
EP = 16 * EPT
RPT = 640
CPR = 160
ACC_N = 16 * RPT


def _prologue_body(x_ref, pos_ref, win_ref, wsrc_ref, wlin_ref, wpos_ref,
                   bin_ref, bpos_ref, table_ref, q_ref):
    h = jax.nn.relu(
        jnp.dot(x_ref[...], win_ref[...], preferred_element_type=jnp.float32)
        + bin_ref[...])
    p = jnp.dot(pos_ref[...], wpos_ref[...], preferred_element_type=jnp.float32)
    b = jnp.dot(h, wsrc_ref[...], preferred_element_type=jnp.float32) + p
    c = jnp.dot(h, wlin_ref[...], preferred_element_type=jnp.float32) - p
    g = jnp.exp(-b)
    table_ref[...] = jnp.concatenate([g, g * c], axis=1).reshape(BLK, 2 * D)
    q_ref[...] = p + bpos_ref[...]


def _epilogue_body(t0_ref, t1_ref, q_ref, wout_ref, bout_ref, o_ref):
    t0 = t0_ref[...]
    agg = (t1_ref[...] + q_ref[...] * t0) / (t0 + 1e-16)
    o_ref[...] = jax.nn.relu(
        jnp.dot(agg, wout_ref[...], preferred_element_type=jnp.float32)
        + bout_ref[...])


def _sc_edge_body(table_hbm, src_hbm, dst_hbm, zeros_hbm, out_hbm,
                  sidx, gidx, didx, rows, buf, acc, sem):
    cid = lax.axis_index("c")
    sid = lax.axis_index("s")

    pltpu.sync_copy(zeros_hbm, buf)
    for k in range(RPT // CPR):
        pltpu.sync_copy(buf, acc.at[pl.ds(sid * RPT + k * CPR, CPR)])
    plsc.subcore_barrier()

    ebase = sid * EPT

    def body(i, carry):
        base = ebase + i * CH
        pltpu.sync_copy(src_hbm.at[pl.ds(base, CH)], sidx)
        pltpu.sync_copy(dst_hbm.at[pl.ds(base, CH)], didx)
        for j in range(CH // 16):
            s = sidx[pl.ds(j * 16, 16)]
            gidx[pl.ds(j * 16, 16)] = s * 2 + cid
        pltpu.async_copy(table_hbm.at[gidx], rows, sem).wait()
        pltpu.sync_copy(rows, acc.at[didx], add=True)
        return carry

    lax.fori_loop(0, NCH, body, 0)
    plsc.subcore_barrier()

    for k in range(RPT // CPR):
        pltpu.sync_copy(acc.at[pl.ds(sid * RPT + k * CPR, CPR)], buf)
        pltpu.sync_copy(buf, out_hbm.at[pl.ds(cid * ACC_N + sid * RPT + k * CPR, CPR)])


@functools.partial(
    pl.kernel,
    out_type=jax.ShapeDtypeStruct((2 * ACC_N, D), jnp.float32),
    mesh=plsc.VectorSubcoreMesh(core_axis_name="c", subcore_axis_name="s"),
    scratch_types=[
        pltpu.VMEM((CH,), jnp.int32),
        pltpu.VMEM((CH,), jnp.int32),
        pltpu.VMEM((CH,), jnp.int32),
        pltpu.VMEM((CH, D), jnp.float32),
        pltpu.VMEM((CPR, D), jnp.float32),
        pltpu.VMEM_SHARED((ACC_N, D), jnp.float32),
        pltpu.SemaphoreType.DMA,
    ],
)
def _sc_edge_kernel(table_hbm, src_hbm, dst_hbm, zeros_hbm, out_hbm,
                    sidx, gidx, didx, rows, buf, acc, sem):
    _sc_edge_body(table_hbm, src_hbm, dst_hbm, zeros_hbm, out_hbm,
                  sidx, gidx, didx, rows, buf, acc, sem)


def kernel(x, edge_index, pos, W_in, b_in, W_lin, W_src, W_dst, W_pos, b_pos,
           W_out, b_out):
    del W_dst

    posp = jnp.pad(pos, ((0, 0), (0, 8 - pos.shape[1])))
    wpos_t = jnp.pad(W_pos.T, ((0, 8 - W_pos.shape[1]), (0, 0)))

    table, q = pl.pallas_call(
        _prologue_body,
        grid=(N // BLK,),
        in_specs=[
            pl.BlockSpec((BLK, D), lambda i: (i, 0)),
            pl.BlockSpec((BLK, 8), lambda i: (i, 0)),
            pl.BlockSpec((D, D), lambda i: (0, 0)),
            pl.BlockSpec((D, D), lambda i: (0, 0)),
            pl.BlockSpec((D, D), lambda i: (0, 0)),
            pl.BlockSpec((8, D), lambda i: (0, 0)),
            pl.BlockSpec((1, D), lambda i: (0, 0)),
            pl.BlockSpec((1, D), lambda i: (0, 0)),
        ],
        out_specs=[
            pl.BlockSpec((BLK, 2 * D), lambda i: (i, 0)),
            pl.BlockSpec((BLK, D), lambda i: (i, 0)),
        ],
        out_shape=[
            jax.ShapeDtypeStruct((N, 2 * D), jnp.float32),
            jax.ShapeDtypeStruct((N, D), jnp.float32),
        ],
    )(x, posp, W_in.T, W_src.T, W_lin.T, wpos_t, b_in.reshape(1, D),
      b_pos.reshape(1, D))

    table2 = jnp.pad(table.reshape(2 * N, D), ((0, 800), (0, 0)))

    src = edge_index[0].astype(jnp.int32)
    dst = edge_index[1].astype(jnp.int32)
    srcp = jnp.pad(src, (0, EP - E))
    dstp = jnp.pad(dst, (0, EP - E), constant_values=N)
    zeros = jnp.zeros((CPR, D), jnp.float32)

    sc_out = _sc_edge_kernel(table2, srcp, dstp, zeros)
    t0 = sc_out[0:N]
    t1 = sc_out[ACC_N:ACC_N + N]

    out = pl.pallas_call(
        _epilogue_body,
        grid=(N // EBK,),
        in_specs=[
            pl.BlockSpec((EBK, D), lambda i: (i, 0)),
            pl.BlockSpec((EBK, D), lambda i: (i, 0)),
            pl.BlockSpec((EBK, D), lambda i: (i, 0)),
            pl.BlockSpec((D, D), lambda i: (0, 0)),
            pl.BlockSpec((1, D), lambda i: (0, 0)),
        ],
        out_specs=pl.BlockSpec((EBK, D), lambda i: (i, 0)),
        out_shape=jax.ShapeDtypeStruct((N, D), jnp.float32),
    )(t0, t1, q, W_out.T, b_out.reshape(1, D))

    return out


# speedup vs baseline: 1.5988x; 1.5073x over previous
import functools

import jax
import jax.numpy as jnp
from jax import lax
from jax.experimental import pallas as pl
from jax.experimental.pallas import tpu as pltpu
from jax.experimental.pallas import tpu_sc as plsc

N = 10000
D = 128
E = 320000

BLK = 400
NT = 10400
CH = 128
NCH = 157
EPT = CH * NCH
EP = 16 * EPT
RPT = 640
CPR = 160
ACC_N = 16 * RPT


def _prologue_body(x_ref, pos_ref, win_ref, wsrc_ref, wlin_ref, wpos_ref,
                   bin_ref, bpos_ref, table_ref, q_ref):
    h = jax.nn.relu(
        jnp.dot(x_ref[...], win_ref[...], preferred_element_type=jnp.float32)
        + bin_ref[...])
    p = jnp.dot(pos_ref[...], wpos_ref[...], preferred_element_type=jnp.float32)
    b = jnp.dot(h, wsrc_ref[...], preferred_element_type=jnp.float32) + p
    c = jnp.dot(h, wlin_ref[...], preferred_element_type=jnp.float32) - p
    g = jnp.exp(-b)
    table_ref[...] = jnp.concatenate([g, g * c], axis=1).reshape(BLK, 2 * D)
    q_ref[...] = p + bpos_ref[...]


def _epilogue_body(t0_ref, t1_ref, q_ref, wout_ref, bout_ref, o_ref):
    t0 = t0_ref[...]
    agg = (t1_ref[...] + q_ref[...] * t0) / (t0 + 1e-16)
    o_ref[...] = jax.nn.relu(
        jnp.dot(agg, wout_ref[...], preferred_element_type=jnp.float32)
        + bout_ref[...])


def _sc_edge_body(table_hbm, src_hbm, dst_hbm, zeros_hbm, out_hbm,
                  sidx, gidx, didx, rows, buf, acc, sem):
    cid = lax.axis_index("c")
    sid = lax.axis_index("s")

    pltpu.sync_copy(zeros_hbm, buf)
    for k in range(RPT // CPR):
        pltpu.sync_copy(buf, acc.at[pl.ds(sid * RPT + k * CPR, CPR)])
    plsc.subcore_barrier()

    def body(i, carry):
        base = sid * EPT + i * CH
        pltpu.sync_copy(src_hbm.at[pl.ds(base, CH)], sidx)
        pltpu.sync_copy(dst_hbm.at[pl.ds(base, CH)], didx)
        for j in range(CH // 16):
            s = sidx[pl.ds(j * 16, 16)]
            gidx[pl.ds(j * 16, 16)] = s * 2 + cid
        pltpu.async_copy(table_hbm.at[gidx], rows, sem).wait()
        pltpu.sync_copy(rows, acc.at[didx], add=True)
        return carry

    lax.fori_loop(0, NCH, body, 0)
    plsc.subcore_barrier()

    for k in range(RPT // CPR):
        pltpu.sync_copy(acc.at[pl.ds(sid * RPT + k * CPR, CPR)], buf)
        pltpu.sync_copy(buf, out_hbm.at[pl.ds(cid * ACC_N + sid * RPT + k * CPR, CPR)])


@functools.partial(
    pl.kernel,
    out_type=jax.ShapeDtypeStruct((2 * ACC_N, D), jnp.float32),
    mesh=plsc.VectorSubcoreMesh(core_axis_name="c", subcore_axis_name="s"),
    scratch_types=[
        pltpu.VMEM((CH,), jnp.int32),
        pltpu.VMEM((CH,), jnp.int32),
        pltpu.VMEM((CH,), jnp.int32),
        pltpu.VMEM((CH, D), jnp.float32),
        pltpu.VMEM((CPR, D), jnp.float32),
        pltpu.VMEM_SHARED((ACC_N, D), jnp.float32),
        pltpu.SemaphoreType.DMA,
    ],
)
def _sc_edge_kernel(table_hbm, src_hbm, dst_hbm, zeros_hbm, out_hbm,
                    sidx, gidx, didx, rows, buf, acc, sem):
    _sc_edge_body(table_hbm, src_hbm, dst_hbm, zeros_hbm, out_hbm,
                  sidx, gidx, didx, rows, buf, acc, sem)


def kernel(x, edge_index, pos, W_in, b_in, W_lin, W_src, W_dst, W_pos, b_pos,
           W_out, b_out):
    del W_dst

    xp = jnp.pad(x, ((0, NT - N), (0, 0)))
    posp = jnp.pad(pos, ((0, NT - N), (0, 8 - pos.shape[1])))
    win_t = W_in.T
    wsrc_t = W_src.T
    wlin_t = W_lin.T
    wpos_t = jnp.pad(W_pos.T, ((0, 8 - W_pos.shape[1]), (0, 0)))
    bin2 = b_in.reshape(1, D)
    bpos2 = b_pos.reshape(1, D)

    grid_pro = NT // BLK
    table, q = pl.pallas_call(
        _prologue_body,
        grid=(grid_pro,),
        in_specs=[
            pl.BlockSpec((BLK, D), lambda i: (i, 0)),
            pl.BlockSpec((BLK, 8), lambda i: (i, 0)),
            pl.BlockSpec((D, D), lambda i: (0, 0)),
            pl.BlockSpec((D, D), lambda i: (0, 0)),
            pl.BlockSpec((D, D), lambda i: (0, 0)),
            pl.BlockSpec((8, D), lambda i: (0, 0)),
            pl.BlockSpec((1, D), lambda i: (0, 0)),
            pl.BlockSpec((1, D), lambda i: (0, 0)),
        ],
        out_specs=[
            pl.BlockSpec((BLK, 2 * D), lambda i: (i, 0)),
            pl.BlockSpec((BLK, D), lambda i: (i, 0)),
        ],
        out_shape=[
            jax.ShapeDtypeStruct((NT, 2 * D), jnp.float32),
            jax.ShapeDtypeStruct((NT, D), jnp.float32),
        ],
    )(xp, posp, win_t, wsrc_t, wlin_t, wpos_t, bin2, bpos2)

    table2 = table.reshape(2 * NT, D)

    src = edge_index[0].astype(jnp.int32)
    dst = edge_index[1].astype(jnp.int32)
    srcp = jnp.pad(src, (0, EP - E))
    dstp = jnp.pad(dst, (0, EP - E), constant_values=N)
    zeros = jnp.zeros((CPR, D), jnp.float32)

    sc_out = _sc_edge_kernel(table2, srcp, dstp, zeros)

    t0 = sc_out[0:N]
    t1 = sc_out[ACC_N:ACC_N + N]
    qn = q[0:N]

    grid_epi = N // BLK
    out = pl.pallas_call(
        _epilogue_body,
        grid=(grid_epi,),
        in_specs=[
            pl.BlockSpec((BLK, D), lambda i: (i, 0)),
            pl.BlockSpec((BLK, D), lambda i: (i, 0)),
            pl.BlockSpec((BLK, D), lambda i: (i, 0)),
            pl.BlockSpec((D, D), lambda i: (0, 0)),
            pl.BlockSpec((1, D), lambda i: (0, 0)),
        ],
        out_specs=pl.BlockSpec((BLK, D), lambda i: (i, 0)),
        out_shape=jax.ShapeDtypeStruct((N, D), jnp.float32),
    )(t0, t1, qn, W_out.T, b_out.reshape(1, D))

    return out
